# Initial kernel scaffold; baseline (speedup 1.0000x reference)
import jax
import jax.numpy as jnp
from jax.experimental import pallas as pl


def kernel(x, edge_index, batch, W_gcn, b_gcn, W_lin, b_lin):
    raise NotImplementedError("write your pallas kernel here")


# speedup vs baseline: 78.2696x; 78.2696x over previous
import functools

import jax
import jax.numpy as jnp
from jax import lax
from jax.experimental import pallas as pl
from jax.experimental.pallas import tpu as pltpu
from jax.experimental.pallas import tpu_sc as plsc

NC = 2
NS = 16
L = 16
G = 64


def _newton_rsqrt(d):
    i = lax.bitcast_convert_type(d, jnp.int32)
    i = jnp.int32(0x5F3759DF) - lax.shift_right_logical(i, 1)
    y = lax.bitcast_convert_type(i, jnp.float32)
    for _ in range(3):
        y = y * (1.5 - 0.5 * d * y * y)
    return y


def _sc_body(npad, erows, src_hbm, dst_hbm, batch_hbm, c_out, cnt_out,
             zbuf, onesb, dstb, srcb, degb, dinv640, batchb, dinvb,
             fidx, wval, sfi, sfv, cbi, cbv,
             sc_deg, sc_dinv, sc_c, sc_cnt):
    c = lax.axis_index("c")
    s = lax.axis_index("s")
    wid = s * NC + c

    nslice = npad // NS
    crows_tile = erows // (NC * NS)
    drows_tile = erows // NS
    cstripe = npad * G // NS
    zlen = zbuf.shape[0]

    def zero_loop(i, _):
        zbuf[pl.ds(i * L, L)] = jnp.zeros((L,), jnp.float32)
        return 0

    lax.fori_loop(0, zlen // L, zero_loop, 0)
    for i in range(8):
        onesb[pl.ds(i * L, L)] = jnp.ones((L,), jnp.float32)

    if True:
        pltpu.sync_copy(zbuf.at[pl.ds(0, nslice)], sc_deg.at[pl.ds(s * nslice, nslice)])
        for k in range(cstripe // zlen):
            pltpu.sync_copy(zbuf, sc_c.at[pl.ds(s * cstripe + k * zlen, zlen)])

        @pl.when(jnp.logical_and(c == 0, s == 0))
        def _():
            pltpu.sync_copy(zbuf.at[pl.ds(0, G)], sc_cnt)

        plsc.subcore_barrier()

        pltpu.sync_copy(dst_hbm.at[pl.ds(s * drows_tile, drows_tile)], dstb)

        def deg_loop(j, _):
            pltpu.sync_copy(onesb, sc_deg.at[dstb.at[j]], add=True)
            return 0

        lax.fori_loop(0, drows_tile, deg_loop, 0)
        plsc.subcore_barrier()

        pltpu.sync_copy(sc_deg.at[pl.ds(s * nslice, nslice)], degb)

        def dinv_loop(i, _):
            d = degb[pl.ds(i * L, L)] + 1.0
            y = _newton_rsqrt(d)
            node = s * nslice + i * L + lax.iota(jnp.int32, L)
            y = jnp.where(node < N_REAL, y, 0.0)
            dinv640[pl.ds(i * L, L)] = y
            return 0

        lax.fori_loop(0, nslice // L, dinv_loop, 0)
        pltpu.sync_copy(dinv640, sc_dinv.at[pl.ds(s * nslice, nslice)])
        plsc.subcore_barrier()

        pltpu.sync_copy(sc_dinv, dinvb)
        pltpu.sync_copy(batch_hbm, batchb)
        pltpu.sync_copy(src_hbm.at[pl.ds(wid * crows_tile, crows_tile)], srcb)
        pltpu.sync_copy(dst_hbm.at[pl.ds(wid * crows_tile, crows_tile)],
                        dstb.at[pl.ds(0, crows_tile)])

        def edge_loop(j, _):
            for k in range(128 // L):
                src16 = srcb[j, pl.ds(k * L, L)]
                dst16 = dstb[j, pl.ds(k * L, L)]
                b16 = plsc.load_gather(batchb, [dst16])
                wd = plsc.load_gather(dinvb, [dst16])
                ws = plsc.load_gather(dinvb, [src16])
                fidx[j, pl.ds(k * L, L)] = lax.shift_left(src16, 6) + b16
                wval[j, pl.ds(k * L, L)] = ws * wd
            return 0

        lax.fori_loop(0, crows_tile, edge_loop, 0)

        def cstream_loop(j, _):
            pltpu.sync_copy(wval.at[j], sc_c.at[fidx.at[j]], add=True)
            return 0

        lax.fori_loop(0, crows_tile, cstream_loop, 0)

        @pl.when(c == 0)
        def _():
            def self_loop(i, _):
                node = s * nslice + i * L + lax.iota(jnp.int32, L)
                b16 = batchb[pl.ds(s * nslice + i * L, L)]
                d16 = dinv640[pl.ds(i * L, L)]
                r = i // 8
                col = (i % 8) * L
                sfi[r, pl.ds(col, L)] = lax.shift_left(node, 6) + b16
                sfv[r, pl.ds(col, L)] = d16 * d16
                cbi[r, pl.ds(col, L)] = b16
                cbv[r, pl.ds(col, L)] = jnp.where(node < N_REAL, 1.0, 0.0)
                return 0

            lax.fori_loop(0, nslice // L, self_loop, 0)

            def sstream_loop(j, _):
                pltpu.sync_copy(sfv.at[j], sc_c.at[sfi.at[j]], add=True)
                pltpu.sync_copy(cbv.at[j], sc_cnt.at[cbi.at[j]], add=True)
                return 0

            lax.fori_loop(0, nslice // 128, sstream_loop, 0)

        plsc.subcore_barrier()

        pltpu.sync_copy(sc_c.at[pl.ds(s * cstripe, cstripe)],
                        c_out.at[c, pl.ds(s * cstripe, cstripe)])

        @pl.when(jnp.logical_and(c == 0, s == 0))
        def _():
            pltpu.sync_copy(sc_cnt, cnt_out)


N_REAL = 10000


def _tc_body(nblocks, x_ref, c_ref, wg_ref, biasp_ref, wl_ref, bl_ref,
             out_ref, acc):
    i = pl.program_id(0)

    @pl.when(i == 0)
    def _():
        acc[...] = jnp.zeros_like(acc)

    S = c_ref[0] + c_ref[1]
    acc[...] += lax.dot_general(S, x_ref[...], (((0,), (0,)), ((), ())),
                                preferred_element_type=jnp.float32)

    @pl.when(i == nblocks - 1)
    def _():
        P = acc[...]
        g = jnp.dot(P, wg_ref[...], preferred_element_type=jnp.float32)
        g = g + biasp_ref[...]
        o = jnp.dot(g, wl_ref[...], preferred_element_type=jnp.float32)
        out_ref[...] = o + bl_ref[...]


def kernel(x, edge_index, batch, W_gcn, b_gcn, W_lin, b_lin):
    n, d = x.shape
    e = edge_index.shape[1]
    assert n == N_REAL

    npad = ((n + NS * L * NC - 1) // (NS * L * NC)) * NS * L * NC
    epr = 128 * 8 * NC * NS
    epad = ((e + epr - 1) // epr) * epr
    erows = epad // 128

    src = edge_index[0].astype(jnp.int32)
    dst = edge_index[1].astype(jnp.int32)
    pad_e = jnp.full((epad - e,), n, jnp.int32)
    srcp = jnp.concatenate([src, pad_e]).reshape(erows, 128)
    dstp = jnp.concatenate([dst, pad_e]).reshape(erows, 128)
    batchp = jnp.concatenate(
        [batch.astype(jnp.int32), jnp.zeros((npad - n,), jnp.int32)])

    mesh = plsc.VectorSubcoreMesh(core_axis_name="c", subcore_axis_name="s",
                                  num_cores=NC, num_subcores=NS)
    vm = pltpu.VMEM
    vs = pltpu.MemorySpace.VMEM_SHARED @ mesh
    sc = pl.kernel(
        functools.partial(_sc_body, npad, erows),
        out_type=(
            jax.ShapeDtypeStruct((NC, npad * G), jnp.float32),
            jax.ShapeDtypeStruct((G,), jnp.float32),
        ),
        mesh=mesh,
        compiler_params=pltpu.CompilerParams(needs_layout_passes=False,
                                             use_tc_tiling_on_sc=False),
        scratch_types=[
            vm((8192,), jnp.float32),
            vm((128,), jnp.float32),
            vm((erows // NS, 128), jnp.int32),
            vm((erows // (NC * NS), 128), jnp.int32),
            vm((npad // NS,), jnp.float32),
            vm((npad // NS,), jnp.float32),
            vm((npad,), jnp.int32),
            vm((npad,), jnp.float32),
            vm((erows // (NC * NS), 128), jnp.int32),
            vm((erows // (NC * NS), 128), jnp.float32),
            vm((npad // NS // 128, 128), jnp.int32),
            vm((npad // NS // 128, 128), jnp.float32),
            vm((npad // NS // 128, 128), jnp.int32),
            vm((npad // NS // 128, 128), jnp.float32),
            vs((npad,), jnp.float32),
            vs((npad,), jnp.float32),
            vs((npad * G,), jnp.float32),
            vs((G,), jnp.float32),
        ],
    )
    c_flat, counts = sc(srcp, dstp, batchp)
    c_parts = c_flat.reshape(NC, npad, G)

    xp = jnp.concatenate([x, jnp.zeros((npad - n, d), x.dtype)])
    biasp = counts[:, None] * b_gcn[None, :]

    bn = 2048
    nblocks = npad // bn
    out = pl.pallas_call(
        functools.partial(_tc_body, nblocks),
        grid=(nblocks,),
        in_specs=[
            pl.BlockSpec((bn, d), lambda i: (i, 0)),
            pl.BlockSpec((NC, bn, G), lambda i: (0, i, 0)),
            pl.BlockSpec((d, d), lambda i: (0, 0)),
            pl.BlockSpec((G, d), lambda i: (0, 0)),
            pl.BlockSpec((d, d), lambda i: (0, 0)),
            pl.BlockSpec((1, d), lambda i: (0, 0)),
        ],
        out_specs=pl.BlockSpec((G, d), lambda i: (0, 0)),
        out_shape=jax.ShapeDtypeStruct((G, d), jnp.float32),
        scratch_shapes=[pltpu.VMEM((G, d), jnp.float32)],
    )(xp, c_parts, W_gcn, biasp, W_lin, b_lin.reshape(1, d))
    return out


# speedup vs baseline: 86.7693x; 1.1086x over previous
import functools

import jax
import jax.numpy as jnp
from jax import lax
from jax.experimental import pallas as pl
from jax.experimental.pallas import tpu as pltpu
from jax.experimental.pallas import tpu_sc as plsc

NC = 2
NS = 16
L = 16
G = 64


def _newton_rsqrt(d):
    i = lax.bitcast_convert_type(d, jnp.int32)
    i = jnp.int32(0x5F3759DF) - lax.shift_right_logical(i, 1)
    y = lax.bitcast_convert_type(i, jnp.float32)
    for _ in range(3):
        y = y * (1.5 - 0.5 * d * y * y)
    return y


def _sc_body(npad, epad, src_hbm, dst_hbm, batch_hbm, c_out, cnt_out,
             zbuf, onesb, dstb, srcb, dstc, degb, dinv640, batchb, dinvb,
             fidx, wval, sfi, sfv, cbi, cbv,
             sc_deg, sc_dinv, sc_c, sc_cnt, sem_a, sem_b, sem_z):
    c = lax.axis_index("c")
    s = lax.axis_index("s")
    wid = s * NC + c

    nslice = npad // NS
    ce_tile = epad // (NC * NS)
    de_tile = epad // NS
    cstripe = npad * G // NS
    zlen = zbuf.shape[0]

    mid = s * NC + (1 - c)

    cp_dd = pltpu.async_copy(dst_hbm.at[pl.ds(mid * ce_tile, ce_tile)],
                             dstb, sem_a)
    cp_src = pltpu.async_copy(src_hbm.at[pl.ds(wid * ce_tile, ce_tile)],
                              srcb, sem_b)
    cp_dstc = pltpu.async_copy(dst_hbm.at[pl.ds(wid * ce_tile, ce_tile)],
                               dstc, sem_a)
    cp_batch = pltpu.async_copy(batch_hbm, batchb, sem_b)

    def zero_loop(i, _):
        zbuf[pl.ds(i * L, L)] = jnp.zeros((L,), jnp.float32)
        return 0

    lax.fori_loop(0, zlen // L, zero_loop, 0)

    def ones_loop(i, _):
        onesb[pl.ds(i * L, L)] = jnp.ones((L,), jnp.float32)
        return 0

    lax.fori_loop(0, ce_tile // L, ones_loop, 0)

    pltpu.sync_copy(zbuf.at[pl.ds(0, nslice)],
                    sc_deg.at[pl.ds(s * nslice, nslice)])

    @pl.when(jnp.logical_and(c == 0, s == 0))
    def _():
        pltpu.sync_copy(zbuf.at[pl.ds(0, G)], sc_cnt)

    zcp = [pltpu.async_copy(zbuf,
                            sc_c.at[pl.ds(s * cstripe + k * zlen, zlen)],
                            sem_z)
           for k in range(cstripe // zlen)]
    cp_dd.wait()
    cp_dstc.wait()
    plsc.subcore_barrier()

    d1 = pltpu.async_copy(onesb, sc_deg.at[dstb], sem_a, add=True)
    d2 = pltpu.async_copy(onesb, sc_deg.at[dstc], sem_a, add=True)
    d1.wait()
    d2.wait()
    plsc.subcore_barrier()

    pltpu.sync_copy(sc_deg.at[pl.ds(s * nslice, nslice)], degb)

    def dinv_loop(i, _):
        d = degb[pl.ds(i * L, L)] + 1.0
        y = _newton_rsqrt(d)
        node = s * nslice + i * L + lax.iota(jnp.int32, L)
        y = jnp.where(node < N_REAL, y, 0.0)
        dinv640[pl.ds(i * L, L)] = y
        return 0

    lax.fori_loop(0, nslice // L, dinv_loop, 0)
    pltpu.sync_copy(dinv640, sc_dinv.at[pl.ds(s * nslice, nslice)])
    for cp in zcp:
        cp.wait()
    plsc.subcore_barrier()

    pltpu.sync_copy(sc_dinv, dinvb)
    cp_batch.wait()
    cp_src.wait()

    def edge_loop(j, _):
        for k in range(8):
            o = j * (8 * L) + k * L
            src16 = srcb[pl.ds(o, L)]
            dst16 = dstc[pl.ds(o, L)]
            b16 = plsc.load_gather(batchb, [dst16])
            wd = plsc.load_gather(dinvb, [dst16])
            ws = plsc.load_gather(dinvb, [src16])
            fidx[pl.ds(o, L)] = lax.shift_left(src16, 6) + b16
            wval[pl.ds(o, L)] = ws * wd
        return 0

    lax.fori_loop(0, ce_tile // (8 * L), edge_loop, 0)
    c_cp = pltpu.async_copy(wval, sc_c.at[fidx], sem_a, add=True)

    @pl.when(c == 0)
    def _():
        def self_loop(i, _):
            node = s * nslice + i * L + lax.iota(jnp.int32, L)
            b16 = batchb[pl.ds(s * nslice + i * L, L)]
            d16 = dinv640[pl.ds(i * L, L)]
            sfi[pl.ds(i * L, L)] = lax.shift_left(node, 6) + b16
            sfv[pl.ds(i * L, L)] = d16 * d16
            cbi[pl.ds(i * L, L)] = b16
            cbv[pl.ds(i * L, L)] = jnp.where(node < N_REAL, 1.0, 0.0)
            return 0

        lax.fori_loop(0, nslice // L, self_loop, 0)
        pltpu.async_copy(sfv, sc_c.at[sfi], sem_z, add=True).wait()
        pltpu.async_copy(cbv, sc_cnt.at[cbi], sem_z, add=True).wait()

    c_cp.wait()
    plsc.subcore_barrier()

    pltpu.sync_copy(sc_c.at[pl.ds(s * cstripe, cstripe)],
                    c_out.at[c, pl.ds(s * cstripe, cstripe)])

    @pl.when(jnp.logical_and(c == 0, s == 0))
    def _():
        pltpu.sync_copy(sc_cnt, cnt_out)


N_REAL = 10000


def _tc_body(nblocks, x_ref, c_ref, wg_ref, biasp_ref, wl_ref, bl_ref,
             out_ref, acc):
    i = pl.program_id(0)

    @pl.when(i == 0)
    def _():
        acc[...] = jnp.zeros_like(acc)

    S = c_ref[0] + c_ref[1]
    acc[...] += lax.dot_general(S, x_ref[...], (((0,), (0,)), ((), ())),
                                preferred_element_type=jnp.float32)

    @pl.when(i == nblocks - 1)
    def _():
        P = acc[...]
        g = jnp.dot(P, wg_ref[...], preferred_element_type=jnp.float32)
        g = g + biasp_ref[...]
        o = jnp.dot(g, wl_ref[...], preferred_element_type=jnp.float32)
        out_ref[...] = o + bl_ref[...]


def kernel(x, edge_index, batch, W_gcn, b_gcn, W_lin, b_lin):
    n, d = x.shape
    e = edge_index.shape[1]
    assert n == N_REAL

    npad = ((n + NS * L * NC - 1) // (NS * L * NC)) * NS * L * NC
    epr = 128 * 8 * NC * NS
    epad = ((e + epr - 1) // epr) * epr
    erows = epad // 128

    src = edge_index[0].astype(jnp.int32)
    dst = edge_index[1].astype(jnp.int32)
    pad_e = jnp.full((epad - e,), n, jnp.int32)
    srcp = jnp.concatenate([src, pad_e])
    dstp = jnp.concatenate([dst, pad_e])
    batchp = jnp.concatenate(
        [batch.astype(jnp.int32), jnp.zeros((npad - n,), jnp.int32)])

    mesh = plsc.VectorSubcoreMesh(core_axis_name="c", subcore_axis_name="s",
                                  num_cores=NC, num_subcores=NS)
    vm = pltpu.VMEM
    vs = pltpu.MemorySpace.VMEM_SHARED @ mesh
    sc = pl.kernel(
        functools.partial(_sc_body, npad, epad),
        out_type=(
            jax.ShapeDtypeStruct((NC, npad * G), jnp.float32),
            jax.ShapeDtypeStruct((G,), jnp.float32),
        ),
        mesh=mesh,
        compiler_params=pltpu.CompilerParams(needs_layout_passes=False,
                                             use_tc_tiling_on_sc=False),
        scratch_types=[
            vm((2048,), jnp.float32),
            vm((epad // (NC * NS),), jnp.float32),
            vm((epad // (NC * NS),), jnp.int32),
            vm((epad // (NC * NS),), jnp.int32),
            vm((epad // (NC * NS),), jnp.int32),
            vm((npad // NS,), jnp.float32),
            vm((npad // NS,), jnp.float32),
            vm((npad,), jnp.int32),
            vm((npad,), jnp.float32),
            vm((epad // (NC * NS),), jnp.int32),
            vm((epad // (NC * NS),), jnp.float32),
            vm((npad // NS,), jnp.int32),
            vm((npad // NS,), jnp.float32),
            vm((npad // NS,), jnp.int32),
            vm((npad // NS,), jnp.float32),
            vs((npad,), jnp.float32),
            vs((npad,), jnp.float32),
            vs((npad * G,), jnp.float32),
            vs((G,), jnp.float32),
            pltpu.SemaphoreType.DMA,
            pltpu.SemaphoreType.DMA,
            pltpu.SemaphoreType.DMA,
        ],
    )
    c_flat, counts = sc(srcp, dstp, batchp)
    c_parts = c_flat.reshape(NC, npad, G)

    xp = jnp.concatenate([x, jnp.zeros((npad - n, d), x.dtype)])
    biasp = counts[:, None] * b_gcn[None, :]

    bn = 2048
    nblocks = npad // bn
    out = pl.pallas_call(
        functools.partial(_tc_body, nblocks),
        grid=(nblocks,),
        in_specs=[
            pl.BlockSpec((bn, d), lambda i: (i, 0)),
            pl.BlockSpec((NC, bn, G), lambda i: (0, i, 0)),
            pl.BlockSpec((d, d), lambda i: (0, 0)),
            pl.BlockSpec((G, d), lambda i: (0, 0)),
            pl.BlockSpec((d, d), lambda i: (0, 0)),
            pl.BlockSpec((1, d), lambda i: (0, 0)),
        ],
        out_specs=pl.BlockSpec((G, d), lambda i: (0, 0)),
        out_shape=jax.ShapeDtypeStruct((G, d), jnp.float32),
        scratch_shapes=[pltpu.VMEM((G, d), jnp.float32)],
    )(xp, c_parts, W_gcn, biasp, W_lin, b_lin.reshape(1, d))
    return out


# speedup vs baseline: 118.2762x; 1.3631x over previous
import functools

import jax
import jax.numpy as jnp
from jax import lax
from jax.experimental import pallas as pl
from jax.experimental.pallas import tpu as pltpu
from jax.experimental.pallas import tpu_sc as plsc

NC = 2
NS = 16
L = 16
G = 64


def _newton_rsqrt(d):
    i = lax.bitcast_convert_type(d, jnp.int32)
    i = jnp.int32(0x5F3759DF) - lax.shift_right_logical(i, 1)
    y = lax.bitcast_convert_type(i, jnp.float32)
    for _ in range(3):
        y = y * (1.5 - 0.5 * d * y * y)
    return y


def _sc_body(npad, etot, h, src_hbm, dst_hbm, batch_hbm, c_out, cnt_out,
             zbuf, onesb, dstb, srcb, dstc, degb, dinv640, batchb, dinvb,
             fidx, wval, sfi, sfv, cbi, cbv,
             sc_deg, sc_dinv, sc_c, sc_cnt, sem_a, sem_b, sem_z):
    c = lax.axis_index("c")
    s = lax.axis_index("s")
    wid = s * NC + c

    nslice = npad // NS
    ce_tile = etot // (NC * NS)
    cstripe = h * 128 // NS
    zlen = zbuf.shape[0]

    mid = s * NC + (1 - c)

    cp_dd = pltpu.async_copy(dst_hbm.at[pl.ds(mid * ce_tile, ce_tile)],
                             dstb, sem_a)
    cp_src = pltpu.async_copy(src_hbm.at[pl.ds(wid * ce_tile, ce_tile)],
                              srcb, sem_b)
    cp_dstc = pltpu.async_copy(dst_hbm.at[pl.ds(wid * ce_tile, ce_tile)],
                               dstc, sem_a)
    cp_batch = pltpu.async_copy(batch_hbm, batchb, sem_b)

    def col_split(j16, b16):
        m = j16 < h
        return (lax.shift_left(j16 - jnp.where(m, 0, h), 7)
                + jnp.where(m, 0, G) + b16)

    def zero_loop(i, _):
        zbuf[pl.ds(i * L, L)] = jnp.zeros((L,), jnp.float32)
        return 0

    lax.fori_loop(0, zlen // L, zero_loop, 0)

    def ones_loop(i, _):
        onesb[pl.ds(i * L, L)] = jnp.ones((L,), jnp.float32)
        return 0

    lax.fori_loop(0, ce_tile // L, ones_loop, 0)

    pltpu.sync_copy(zbuf.at[pl.ds(0, nslice)],
                    sc_deg.at[pl.ds(s * nslice, nslice)])

    @pl.when(jnp.logical_and(c == 0, s == 0))
    def _():
        pltpu.sync_copy(zbuf.at[pl.ds(0, G)], sc_cnt)

    zcp = []
    off = 0
    while off < cstripe:
        step = min(zlen, cstripe - off)
        zcp.append(pltpu.async_copy(
            zbuf.at[pl.ds(0, step)],
            sc_c.at[pl.ds(s * cstripe + off, step)], sem_z))
        off += step
    cp_dd.wait()
    cp_dstc.wait()
    plsc.subcore_barrier()

    d1 = pltpu.async_copy(onesb, sc_deg.at[dstb], sem_a, add=True)
    d2 = pltpu.async_copy(onesb, sc_deg.at[dstc], sem_a, add=True)
    d1.wait()
    d2.wait()
    plsc.subcore_barrier()

    pltpu.sync_copy(sc_deg.at[pl.ds(s * nslice, nslice)], degb)

    def dinv_loop(i, _):
        d = degb[pl.ds(i * L, L)] + 1.0
        y = _newton_rsqrt(d)
        node = s * nslice + i * L + lax.iota(jnp.int32, L)
        y = jnp.where(node < N_REAL, y, 0.0)
        dinv640[pl.ds(i * L, L)] = y
        return 0

    lax.fori_loop(0, nslice // L, dinv_loop, 0)
    pltpu.sync_copy(dinv640, sc_dinv.at[pl.ds(s * nslice, nslice)])
    for cp in zcp:
        cp.wait()
    plsc.subcore_barrier()

    pltpu.sync_copy(sc_dinv, dinvb)
    cp_batch.wait()
    cp_src.wait()

    def edge_loop(j, _):
        for k in range(5):
            o = j * (5 * L) + k * L
            src16 = srcb[pl.ds(o, L)]
            dst16 = dstc[pl.ds(o, L)]
            b16 = plsc.load_gather(batchb, [dst16])
            wd = plsc.load_gather(dinvb, [dst16])
            ws = plsc.load_gather(dinvb, [src16])
            fidx[pl.ds(o, L)] = col_split(src16, b16)
            wval[pl.ds(o, L)] = ws * wd
        return 0

    lax.fori_loop(0, ce_tile // (5 * L), edge_loop, 0)
    c_cp = pltpu.async_copy(wval, sc_c.at[fidx], sem_a, add=True)

    @pl.when(c == 0)
    def _():
        def self_loop(i, _):
            node = s * nslice + i * L + lax.iota(jnp.int32, L)
            b16 = batchb[pl.ds(s * nslice + i * L, L)]
            d16 = dinv640[pl.ds(i * L, L)]
            real = node < N_REAL
            sfi[pl.ds(i * L, L)] = jnp.where(real, col_split(node, b16), 0)
            sfv[pl.ds(i * L, L)] = d16 * d16
            cbi[pl.ds(i * L, L)] = b16
            cbv[pl.ds(i * L, L)] = jnp.where(real, 1.0, 0.0)
            return 0

        lax.fori_loop(0, nslice // L, self_loop, 0)
        pltpu.async_copy(sfv, sc_c.at[sfi], sem_z, add=True).wait()
        pltpu.async_copy(cbv, sc_cnt.at[cbi], sem_z, add=True).wait()

    c_cp.wait()
    plsc.subcore_barrier()

    pltpu.sync_copy(sc_c.at[pl.ds(s * cstripe, cstripe)],
                    c_out.at[pl.ds((c * NS + s) * cstripe, cstripe)])

    @pl.when(jnp.logical_and(c == 0, s == 0))
    def _():
        pltpu.sync_copy(sc_cnt, cnt_out)


N_REAL = 10000


def _tc_body(nblocks, xl_ref, xr_ref, c_ref, wg_ref, biasp_ref, wl_ref,
             bl_ref, out_ref, acc):
    i = pl.program_id(0)

    @pl.when(i == 0)
    def _():
        acc[...] = jnp.zeros_like(acc)

    S = c_ref[0] + c_ref[1]
    dn = (((0,), (0,)), ((), ()))
    acc[...] += (
        lax.dot_general(S[:, :G], xl_ref[...], dn,
                        preferred_element_type=jnp.float32)
        + lax.dot_general(S[:, G:], xr_ref[...], dn,
                          preferred_element_type=jnp.float32))

    @pl.when(i == nblocks - 1)
    def _():
        P = acc[...]
        g = jnp.dot(P, wg_ref[...], preferred_element_type=jnp.float32)
        g = g + biasp_ref[...]
        o = jnp.dot(g, wl_ref[...], preferred_element_type=jnp.float32)
        out_ref[...] = o + bl_ref[...]


def kernel(x, edge_index, batch, W_gcn, b_gcn, W_lin, b_lin):
    n, d = x.shape
    e = edge_index.shape[1]
    assert n == N_REAL and e % (80 * NC * NS) == 0
    h = n // 2

    npad = ((n + NS * L * NC - 1) // (NS * L * NC)) * NS * L * NC

    src = edge_index[0].astype(jnp.int32)
    dst = edge_index[1].astype(jnp.int32)
    batchp = jnp.concatenate(
        [batch.astype(jnp.int32), jnp.zeros((npad - n,), jnp.int32)])

    mesh = plsc.VectorSubcoreMesh(core_axis_name="c", subcore_axis_name="s",
                                  num_cores=NC, num_subcores=NS)
    vm = pltpu.VMEM
    vs = pltpu.MemorySpace.VMEM_SHARED @ mesh
    sc = pl.kernel(
        functools.partial(_sc_body, npad, e, h),
        out_type=(
            jax.ShapeDtypeStruct((NC * h * 128,), jnp.float32),
            jax.ShapeDtypeStruct((G,), jnp.float32),
        ),
        mesh=mesh,
        compiler_params=pltpu.CompilerParams(needs_layout_passes=False,
                                             use_tc_tiling_on_sc=False),
        scratch_types=[
            vm((2048,), jnp.float32),
            vm((e // (NC * NS),), jnp.float32),
            vm((e // (NC * NS),), jnp.int32),
            vm((e // (NC * NS),), jnp.int32),
            vm((e // (NC * NS),), jnp.int32),
            vm((npad // NS,), jnp.float32),
            vm((npad // NS,), jnp.float32),
            vm((npad,), jnp.int32),
            vm((npad,), jnp.float32),
            vm((e // (NC * NS),), jnp.int32),
            vm((e // (NC * NS),), jnp.float32),
            vm((npad // NS,), jnp.int32),
            vm((npad // NS,), jnp.float32),
            vm((npad // NS,), jnp.int32),
            vm((npad // NS,), jnp.float32),
            vs((npad,), jnp.float32),
            vs((npad,), jnp.float32),
            vs((h * 128,), jnp.float32),
            vs((G,), jnp.float32),
            pltpu.SemaphoreType.DMA,
            pltpu.SemaphoreType.DMA,
            pltpu.SemaphoreType.DMA,
        ],
    )
    c_flat, counts = sc(src, dst, batchp)
    c_parts = c_flat.reshape(NC, h, 128)

    biasp = counts[:, None] * b_gcn[None, :]

    bn = 1000
    nblocks = h // bn
    out = pl.pallas_call(
        functools.partial(_tc_body, nblocks),
        grid=(nblocks,),
        in_specs=[
            pl.BlockSpec((bn, d), lambda i: (i, 0)),
            pl.BlockSpec((bn, d), lambda i: (i + nblocks, 0)),
            pl.BlockSpec((NC, bn, 128), lambda i: (0, i, 0)),
            pl.BlockSpec((d, d), lambda i: (0, 0)),
            pl.BlockSpec((G, d), lambda i: (0, 0)),
            pl.BlockSpec((d, d), lambda i: (0, 0)),
            pl.BlockSpec((1, d), lambda i: (0, 0)),
        ],
        out_specs=pl.BlockSpec((G, d), lambda i: (0, 0)),
        out_shape=jax.ShapeDtypeStruct((G, d), jnp.float32),
        scratch_shapes=[pltpu.VMEM((G, d), jnp.float32)],
    )(x, x, c_parts, W_gcn, biasp, W_lin, b_lin.reshape(1, d))
    return out


# speedup vs baseline: 120.0752x; 1.0152x over previous
import functools

import jax
import jax.numpy as jnp
from jax import lax
from jax.experimental import pallas as pl
from jax.experimental.pallas import tpu as pltpu
from jax.experimental.pallas import tpu_sc as plsc

NC = 2
NS = 16
L = 16
G = 64


def _newton_rsqrt(d):
    i = lax.bitcast_convert_type(d, jnp.int32)
    i = jnp.int32(0x5F3759DF) - lax.shift_right_logical(i, 1)
    y = lax.bitcast_convert_type(i, jnp.float32)
    for _ in range(3):
        y = y * (1.5 - 0.5 * d * y * y)
    return y


def _sc_body(npad, etot, h, src_hbm, dst_hbm, batch_hbm, c_out, cnt_out,
             zbuf, onesb, dstb, srcb, dstc, degb, dinv640, batchb, dinvb,
             fidx, wval, sfi, sfv, cbi, cbv,
             sc_deg, sc_dinv, sc_c, sc_cnt, sem_a, sem_b, sem_z):
    c = lax.axis_index("c")
    s = lax.axis_index("s")
    wid = s * NC + c

    nslice = npad // NS
    ce_tile = etot // (NC * NS)
    cstripe = h * 128 // NS
    zlen = zbuf.shape[0]

    mid = s * NC + (1 - c)

    cp_dd = pltpu.async_copy(dst_hbm.at[pl.ds(mid * ce_tile, ce_tile)],
                             dstb, sem_a)
    cp_src = pltpu.async_copy(src_hbm.at[pl.ds(wid * ce_tile, ce_tile)],
                              srcb, sem_b)
    cp_dstc = pltpu.async_copy(dst_hbm.at[pl.ds(wid * ce_tile, ce_tile)],
                               dstc, sem_a)
    cp_batch = pltpu.async_copy(batch_hbm, batchb, sem_b)

    def col_split(j16, b16):
        m = j16 < h
        return (lax.shift_left(j16 - jnp.where(m, 0, h), 7)
                + jnp.where(m, 0, G) + b16)

    def zero_loop(i, _):
        zbuf[pl.ds(i * L, L)] = jnp.zeros((L,), jnp.float32)
        return 0

    lax.fori_loop(0, zlen // L, zero_loop, 0)

    def ones_loop(i, _):
        onesb[pl.ds(i * L, L)] = jnp.ones((L,), jnp.float32)
        return 0

    lax.fori_loop(0, ce_tile // L, ones_loop, 0)

    pltpu.sync_copy(zbuf.at[pl.ds(0, nslice)],
                    sc_deg.at[pl.ds(s * nslice, nslice)])

    @pl.when(jnp.logical_and(c == 0, s == 0))
    def _():
        pltpu.sync_copy(zbuf.at[pl.ds(0, G)], sc_cnt)

    zcp = []
    off = 0
    while off < cstripe:
        step = min(zlen, cstripe - off)
        zcp.append(pltpu.async_copy(
            zbuf.at[pl.ds(0, step)],
            sc_c.at[pl.ds(s * cstripe + off, step)], sem_z))
        off += step
    cp_dd.wait()
    cp_dstc.wait()
    plsc.subcore_barrier()

    d1 = pltpu.async_copy(onesb, sc_deg.at[dstb], sem_a, add=True)
    d2 = pltpu.async_copy(onesb, sc_deg.at[dstc], sem_a, add=True)
    cp_batch.wait()
    cp_src.wait()

    def idx_loop(j, _):
        for k in range(5):
            o = j * (5 * L) + k * L
            src16 = srcb[pl.ds(o, L)]
            dst16 = dstc[pl.ds(o, L)]
            b16 = plsc.load_gather(batchb, [dst16])
            fidx[pl.ds(o, L)] = col_split(src16, b16)
        return 0

    lax.fori_loop(0, ce_tile // (5 * L), idx_loop, 0)
    d1.wait()
    d2.wait()
    plsc.subcore_barrier()

    pltpu.sync_copy(sc_deg.at[pl.ds(s * nslice, nslice)], degb)

    def dinv_loop(i, _):
        d = degb[pl.ds(i * L, L)] + 1.0
        y = _newton_rsqrt(d)
        node = s * nslice + i * L + lax.iota(jnp.int32, L)
        y = jnp.where(node < N_REAL, y, 0.0)
        dinv640[pl.ds(i * L, L)] = y
        return 0

    lax.fori_loop(0, nslice // L, dinv_loop, 0)
    pltpu.sync_copy(dinv640, sc_dinv.at[pl.ds(s * nslice, nslice)])
    for cp in zcp:
        cp.wait()
    plsc.subcore_barrier()

    pltpu.sync_copy(sc_dinv, dinvb)

    def edge_loop(j, _):
        for k in range(5):
            o = j * (5 * L) + k * L
            src16 = srcb[pl.ds(o, L)]
            dst16 = dstc[pl.ds(o, L)]
            wd = plsc.load_gather(dinvb, [dst16])
            ws = plsc.load_gather(dinvb, [src16])
            wval[pl.ds(o, L)] = ws * wd
        return 0

    lax.fori_loop(0, ce_tile // (5 * L), edge_loop, 0)
    c_cp = pltpu.async_copy(wval, sc_c.at[fidx], sem_a, add=True)

    @pl.when(c == 0)
    def _():
        def self_loop(i, _):
            node = s * nslice + i * L + lax.iota(jnp.int32, L)
            b16 = batchb[pl.ds(s * nslice + i * L, L)]
            d16 = dinv640[pl.ds(i * L, L)]
            real = node < N_REAL
            sfi[pl.ds(i * L, L)] = jnp.where(real, col_split(node, b16), 0)
            sfv[pl.ds(i * L, L)] = d16 * d16
            cbi[pl.ds(i * L, L)] = b16
            cbv[pl.ds(i * L, L)] = jnp.where(real, 1.0, 0.0)
            return 0

        lax.fori_loop(0, nslice // L, self_loop, 0)
        pltpu.async_copy(sfv, sc_c.at[sfi], sem_z, add=True).wait()
        pltpu.async_copy(cbv, sc_cnt.at[cbi], sem_z, add=True).wait()

    c_cp.wait()
    plsc.subcore_barrier()

    pltpu.sync_copy(sc_c.at[pl.ds(s * cstripe, cstripe)],
                    c_out.at[pl.ds((c * NS + s) * cstripe, cstripe)])

    @pl.when(jnp.logical_and(c == 0, s == 0))
    def _():
        pltpu.sync_copy(sc_cnt, cnt_out)


N_REAL = 10000


def _tc_body(nblocks, xl_ref, xr_ref, c_ref, wg_ref, cnt_ref, bg_ref,
             wl_ref, bl_ref, out_ref, acc):
    i = pl.program_id(0)

    @pl.when(i == 0)
    def _():
        acc[...] = jnp.zeros_like(acc)

    S = c_ref[0] + c_ref[1]
    dn = (((0,), (0,)), ((), ()))
    acc[...] += (
        lax.dot_general(S[:, :G], xl_ref[...], dn,
                        preferred_element_type=jnp.float32)
        + lax.dot_general(S[:, G:], xr_ref[...], dn,
                          preferred_element_type=jnp.float32))

    @pl.when(i == nblocks - 1)
    def _():
        P = acc[...]
        g = jnp.dot(P, wg_ref[...], preferred_element_type=jnp.float32)
        g = g + cnt_ref[...] * bg_ref[...]
        o = jnp.dot(g, wl_ref[...], preferred_element_type=jnp.float32)
        out_ref[...] = o + bl_ref[...]


def kernel(x, edge_index, batch, W_gcn, b_gcn, W_lin, b_lin):
    n, d = x.shape
    e = edge_index.shape[1]
    assert n == N_REAL and e % (80 * NC * NS) == 0
    h = n // 2

    npad = ((n + NS * L * NC - 1) // (NS * L * NC)) * NS * L * NC

    src = edge_index[0].astype(jnp.int32)
    dst = edge_index[1].astype(jnp.int32)
    batchp = jnp.concatenate(
        [batch.astype(jnp.int32), jnp.zeros((npad - n,), jnp.int32)])

    mesh = plsc.VectorSubcoreMesh(core_axis_name="c", subcore_axis_name="s",
                                  num_cores=NC, num_subcores=NS)
    vm = pltpu.VMEM
    vs = pltpu.MemorySpace.VMEM_SHARED @ mesh
    sc = pl.kernel(
        functools.partial(_sc_body, npad, e, h),
        out_type=(
            jax.ShapeDtypeStruct((NC * h * 128,), jnp.float32),
            jax.ShapeDtypeStruct((G,), jnp.float32),
        ),
        mesh=mesh,
        compiler_params=pltpu.CompilerParams(needs_layout_passes=False,
                                             use_tc_tiling_on_sc=False),
        scratch_types=[
            vm((2048,), jnp.float32),
            vm((e // (NC * NS),), jnp.float32),
            vm((e // (NC * NS),), jnp.int32),
            vm((e // (NC * NS),), jnp.int32),
            vm((e // (NC * NS),), jnp.int32),
            vm((npad // NS,), jnp.float32),
            vm((npad // NS,), jnp.float32),
            vm((npad,), jnp.int32),
            vm((npad,), jnp.float32),
            vm((e // (NC * NS),), jnp.int32),
            vm((e // (NC * NS),), jnp.float32),
            vm((npad // NS,), jnp.int32),
            vm((npad // NS,), jnp.float32),
            vm((npad // NS,), jnp.int32),
            vm((npad // NS,), jnp.float32),
            vs((npad,), jnp.float32),
            vs((npad,), jnp.float32),
            vs((h * 128,), jnp.float32),
            vs((G,), jnp.float32),
            pltpu.SemaphoreType.DMA,
            pltpu.SemaphoreType.DMA,
            pltpu.SemaphoreType.DMA,
        ],
    )
    c_flat, counts = sc(src, dst, batchp)
    c_parts = c_flat.reshape(NC, h, 128)

    bn = 1000
    nblocks = h // bn
    out = pl.pallas_call(
        functools.partial(_tc_body, nblocks),
        grid=(nblocks,),
        in_specs=[
            pl.BlockSpec((bn, d), lambda i: (i, 0)),
            pl.BlockSpec((bn, d), lambda i: (i + nblocks, 0)),
            pl.BlockSpec((NC, bn, 128), lambda i: (0, i, 0)),
            pl.BlockSpec((d, d), lambda i: (0, 0)),
            pl.BlockSpec((G, 1), lambda i: (0, 0)),
            pl.BlockSpec((1, d), lambda i: (0, 0)),
            pl.BlockSpec((d, d), lambda i: (0, 0)),
            pl.BlockSpec((1, d), lambda i: (0, 0)),
        ],
        out_specs=pl.BlockSpec((G, d), lambda i: (0, 0)),
        out_shape=jax.ShapeDtypeStruct((G, d), jnp.float32),
        scratch_shapes=[pltpu.VMEM((G, d), jnp.float32)],
    )(x, x, c_parts, W_gcn, counts.reshape(G, 1), b_gcn.reshape(1, d),
      W_lin, b_lin.reshape(1, d))
    return out


# speedup vs baseline: 122.0541x; 1.0165x over previous
import functools

import jax
import jax.numpy as jnp
from jax import lax
from jax.experimental import pallas as pl
from jax.experimental.pallas import tpu as pltpu
from jax.experimental.pallas import tpu_sc as plsc

NC = 2
NS = 16
L = 16
G = 64


def _newton_rsqrt(d):
    i = lax.bitcast_convert_type(d, jnp.int32)
    i = jnp.int32(0x5F3759DF) - lax.shift_right_logical(i, 1)
    y = lax.bitcast_convert_type(i, jnp.float32)
    for _ in range(3):
        y = y * (1.5 - 0.5 * d * y * y)
    return y


def _sc_body(npad, etot, h, src_hbm, dst_hbm, batch_hbm, c_out, cnt_out,
             zbuf, onesb, dstb, srcb, dstc, degb, dinv640, batchb, dinvb,
             fidx, wval, sfi, sfv, cbi, cbv,
             sc_deg, sc_dinv, sc_c, sc_cnt, sem_a, sem_b, sem_z):
    c = lax.axis_index("c")
    s = lax.axis_index("s")
    wid = s * NC + c

    nslice = npad // NS
    ce_tile = etot // (NC * NS)
    cstripe = h * 128 // NS
    zlen = zbuf.shape[0]

    mid = s * NC + (1 - c)

    cp_dd = pltpu.async_copy(dst_hbm.at[pl.ds(mid * ce_tile, ce_tile)],
                             dstb, sem_a)
    cp_src = pltpu.async_copy(src_hbm.at[pl.ds(wid * ce_tile, ce_tile)],
                              srcb, sem_b)
    cp_dstc = pltpu.async_copy(dst_hbm.at[pl.ds(wid * ce_tile, ce_tile)],
                               dstc, sem_a)
    cp_batch = pltpu.async_copy(batch_hbm, batchb, sem_b)

    def col_split(j16, b16):
        m = j16 < h
        return (lax.shift_left(j16 - jnp.where(m, 0, h), 7)
                + jnp.where(m, 0, G) + b16)

    def zero_loop(i, _):
        zbuf[pl.ds(i * L, L)] = jnp.zeros((L,), jnp.float32)
        return 0

    lax.fori_loop(0, zlen // L, zero_loop, 0)

    def ones_loop(i, _):
        onesb[pl.ds(i * L, L)] = jnp.ones((L,), jnp.float32)
        return 0

    lax.fori_loop(0, ce_tile // L, ones_loop, 0)

    pltpu.sync_copy(zbuf.at[pl.ds(0, nslice)],
                    sc_deg.at[pl.ds(s * nslice, nslice)])

    @pl.when(jnp.logical_and(c == 0, s == 0))
    def _():
        pltpu.sync_copy(zbuf.at[pl.ds(0, G)], sc_cnt)

    zcp = []
    off = 0
    while off < cstripe:
        step = min(zlen, cstripe - off)
        zcp.append(pltpu.async_copy(
            zbuf.at[pl.ds(0, step)],
            sc_c.at[pl.ds(s * cstripe + off, step)], sem_z))
        off += step
    cp_dd.wait()
    cp_dstc.wait()
    plsc.subcore_barrier()

    d1 = pltpu.async_copy(onesb, sc_deg.at[dstb], sem_a, add=True)
    d2 = pltpu.async_copy(onesb, sc_deg.at[dstc], sem_a, add=True)
    cp_batch.wait()
    cp_src.wait()

    def idx_loop(j, _):
        for k in range(5):
            o = j * (5 * L) + k * L
            src16 = srcb[pl.ds(o, L)]
            dst16 = dstc[pl.ds(o, L)]
            b16 = plsc.load_gather(batchb, [dst16])
            fidx[pl.ds(o, L)] = col_split(src16, b16)
        return 0

    lax.fori_loop(0, ce_tile // (5 * L), idx_loop, 0)
    d1.wait()
    d2.wait()
    plsc.subcore_barrier()

    pltpu.sync_copy(sc_deg.at[pl.ds(s * nslice, nslice)], degb)

    def dinv_loop(i, _):
        d = degb[pl.ds(i * L, L)] + 1.0
        y = _newton_rsqrt(d)
        node = s * nslice + i * L + lax.iota(jnp.int32, L)
        y = jnp.where(node < N_REAL, y, 0.0)
        dinv640[pl.ds(i * L, L)] = y
        return 0

    lax.fori_loop(0, nslice // L, dinv_loop, 0)
    pltpu.sync_copy(dinv640, sc_dinv.at[pl.ds(s * nslice, nslice)])
    for cp in zcp:
        cp.wait()
    plsc.subcore_barrier()

    pltpu.sync_copy(sc_dinv, dinvb)

    def edge_loop(j, _):
        for k in range(5):
            o = j * (5 * L) + k * L
            src16 = srcb[pl.ds(o, L)]
            dst16 = dstc[pl.ds(o, L)]
            wd = plsc.load_gather(dinvb, [dst16])
            ws = plsc.load_gather(dinvb, [src16])
            wval[pl.ds(o, L)] = ws * wd
        return 0

    lax.fori_loop(0, ce_tile // (5 * L), edge_loop, 0)
    c_cp = pltpu.async_copy(wval, sc_c.at[fidx], sem_a, add=True)

    @pl.when(c == 0)
    def _():
        def self_loop(i, _):
            node = s * nslice + i * L + lax.iota(jnp.int32, L)
            b16 = batchb[pl.ds(s * nslice + i * L, L)]
            d16 = dinv640[pl.ds(i * L, L)]
            real = node < N_REAL
            sfi[pl.ds(i * L, L)] = jnp.where(real, col_split(node, b16), 0)
            sfv[pl.ds(i * L, L)] = d16 * d16
            cbi[pl.ds(i * L, L)] = b16
            cbv[pl.ds(i * L, L)] = jnp.where(real, 1.0, 0.0)
            return 0

        lax.fori_loop(0, nslice // L, self_loop, 0)
        pltpu.async_copy(sfv, sc_c.at[sfi], sem_z, add=True).wait()
        pltpu.async_copy(cbv, sc_cnt.at[cbi], sem_z, add=True).wait()

    c_cp.wait()
    plsc.subcore_barrier()

    pltpu.sync_copy(sc_c.at[pl.ds(s * cstripe, cstripe)],
                    c_out.at[pl.ds((c * NS + s) * cstripe, cstripe)])

    @pl.when(jnp.logical_and(c == 0, s == 0))
    def _():
        pltpu.sync_copy(sc_cnt, cnt_out)


N_REAL = 10000


def _tc_body(nblocks, xl_ref, xr_ref, c_ref, wg_ref, cnt_ref, bg_ref,
             wl_ref, bl_ref, out_ref, acc):
    i = pl.program_id(0)

    @pl.when(i == 0)
    def _():
        acc[...] = jnp.zeros_like(acc)

    S = c_ref[0] + c_ref[1]
    dn = (((0,), (0,)), ((), ()))
    acc[...] += (
        lax.dot_general(S[:, :G], xl_ref[...], dn,
                        preferred_element_type=jnp.float32)
        + lax.dot_general(S[:, G:], xr_ref[...], dn,
                          preferred_element_type=jnp.float32))

    @pl.when(i == nblocks - 1)
    def _():
        P = acc[...]
        g = jnp.dot(P, wg_ref[...], preferred_element_type=jnp.float32)
        g = g + cnt_ref[...] * bg_ref[...]
        o = jnp.dot(g, wl_ref[...], preferred_element_type=jnp.float32)
        out_ref[...] = o + bl_ref[...]


def kernel(x, edge_index, batch, W_gcn, b_gcn, W_lin, b_lin):
    n, d = x.shape
    e = edge_index.shape[1]
    assert n == N_REAL and e % (80 * NC * NS) == 0
    h = n // 2

    npad = ((n + NS * L * NC - 1) // (NS * L * NC)) * NS * L * NC

    src = edge_index[0].astype(jnp.int32)
    dst = edge_index[1].astype(jnp.int32)
    batchp = jnp.concatenate(
        [batch.astype(jnp.int32), jnp.zeros((npad - n,), jnp.int32)])

    mesh = plsc.VectorSubcoreMesh(core_axis_name="c", subcore_axis_name="s",
                                  num_cores=NC, num_subcores=NS)
    vm = pltpu.VMEM
    vs = pltpu.MemorySpace.VMEM_SHARED @ mesh
    sc = pl.kernel(
        functools.partial(_sc_body, npad, e, h),
        out_type=(
            jax.ShapeDtypeStruct((NC * h * 128,), jnp.float32),
            jax.ShapeDtypeStruct((G,), jnp.float32),
        ),
        mesh=mesh,
        compiler_params=pltpu.CompilerParams(needs_layout_passes=False,
                                             use_tc_tiling_on_sc=False),
        scratch_types=[
            vm((2048,), jnp.float32),
            vm((e // (NC * NS),), jnp.float32),
            vm((e // (NC * NS),), jnp.int32),
            vm((e // (NC * NS),), jnp.int32),
            vm((e // (NC * NS),), jnp.int32),
            vm((npad // NS,), jnp.float32),
            vm((npad // NS,), jnp.float32),
            vm((npad,), jnp.int32),
            vm((npad,), jnp.float32),
            vm((e // (NC * NS),), jnp.int32),
            vm((e // (NC * NS),), jnp.float32),
            vm((npad // NS,), jnp.int32),
            vm((npad // NS,), jnp.float32),
            vm((npad // NS,), jnp.int32),
            vm((npad // NS,), jnp.float32),
            vs((npad,), jnp.float32),
            vs((npad,), jnp.float32),
            vs((h * 128,), jnp.float32),
            vs((G,), jnp.float32),
            pltpu.SemaphoreType.DMA,
            pltpu.SemaphoreType.DMA,
            pltpu.SemaphoreType.DMA,
        ],
    )
    c_flat, counts = sc(src, dst, batchp)
    c_parts = c_flat.reshape(NC, h, 128)

    bn = 5000
    nblocks = h // bn
    out = pl.pallas_call(
        functools.partial(_tc_body, nblocks),
        grid=(nblocks,),
        in_specs=[
            pl.BlockSpec((bn, d), lambda i: (i, 0)),
            pl.BlockSpec((bn, d), lambda i: (i + nblocks, 0)),
            pl.BlockSpec((NC, bn, 128), lambda i: (0, i, 0)),
            pl.BlockSpec((d, d), lambda i: (0, 0)),
            pl.BlockSpec((G, 1), lambda i: (0, 0)),
            pl.BlockSpec((1, d), lambda i: (0, 0)),
            pl.BlockSpec((d, d), lambda i: (0, 0)),
            pl.BlockSpec((1, d), lambda i: (0, 0)),
        ],
        out_specs=pl.BlockSpec((G, d), lambda i: (0, 0)),
        out_shape=jax.ShapeDtypeStruct((G, d), jnp.float32),
        scratch_shapes=[pltpu.VMEM((G, d), jnp.float32)],
    )(x, x, c_parts, W_gcn, counts.reshape(G, 1), b_gcn.reshape(1, d),
      W_lin, b_lin.reshape(1, d))
    return out
